# baseline (speedup 1.0000x reference)
import jax
import jax.numpy as jnp
from jax.experimental import pallas as pl
from jax.experimental.pallas import tpu as pltpu

_K = 100
_H = 160
_W = 160
_C = 80
_SCALE = 4.0


def _body(y_ref, score_ref, cls_ref, bcx_ref, bcy_ref,
          whx_ref, why_ref, keep_ref):
    H, W, C = _H, _W, _C
    ninf = jnp.float32(-jnp.inf)

    CH = 40
    pad_row = jnp.full((1, W, C), ninf, dtype=jnp.float32)
    pad_col = jnp.full((CH, 1, C), ninf, dtype=jnp.float32)
    rowmax_parts = []
    for c0 in range(0, H, CH):
        lo = max(c0 - 1, 0)
        hi = min(c0 + CH + 1, H)
        o = c0 - lo
        hmc = y_ref[0, lo:hi, :, :C]
        center = hmc[o:o + CH]
        if lo < c0:
            up = hmc[o - 1:o + CH - 1]
        else:
            up = jnp.concatenate([pad_row, hmc[0:CH - 1]], axis=0)
        if hi > c0 + CH:
            down = hmc[o + 1:o + CH + 1]
        else:
            down = jnp.concatenate([hmc[o + 1:o + CH], pad_row], axis=0)
        vmax = jnp.maximum(center, jnp.maximum(up, down))
        hmax = jnp.maximum(
            vmax,
            jnp.maximum(jnp.concatenate([vmax[:, 1:], pad_col], axis=1),
                        jnp.concatenate([pad_col, vmax[:, :-1]], axis=1)))
        keep_c = jnp.where(center == hmax, center, 0.0)
        keep_ref[c0:c0 + CH] = keep_c
        cm_c = jnp.max(keep_c, axis=2)
        rowmax_parts.append(jnp.max(cm_c, axis=1).reshape(1, CH))
    rowmax = jnp.concatenate(rowmax_parts, axis=1)

    col_iota = (jax.lax.broadcasted_iota(jnp.int32, (W, C), 0) * C
                + jax.lax.broadcasted_iota(jnp.int32, (W, C), 1))
    lane_h = jax.lax.broadcasted_iota(jnp.int32, (1, H), 1)
    lane_c = jax.lax.broadcasted_iota(jnp.int32, (1, 1, C), 2)
    lane_o = jax.lax.broadcasted_iota(jnp.int32, (1, 128), 1)
    big = jnp.int32(1 << 30)

    def step(t, carry):
        rmax, sv, cv, rv, jv = carry
        m = jnp.max(rmax)
        r = jnp.min(jnp.where(rmax == m, lane_h, big))
        row = keep_ref[pl.ds(r, 1)][0]
        c = jnp.min(jnp.where(row == m, col_iota, big))
        k = jnp.mod(c, C)
        j = c // C
        new_row = jnp.where(col_iota == c, -1.0, row)
        keep_ref[pl.ds(r, 1)] = new_row[None]
        rmax = jnp.where(lane_h == r, jnp.max(new_row), rmax)
        sel = lane_o == t
        sv = jnp.where(sel, m, sv)
        cv = jnp.where(sel, k.astype(jnp.float32), cv)
        rv = jnp.where(sel, r, rv)
        jv = jnp.where(sel, j, jv)
        return rmax, sv, cv, rv, jv

    z = jnp.zeros((1, 128), jnp.float32)
    zi = jnp.zeros((1, 128), jnp.int32)
    _, sv, cv, rv, jv = jax.lax.fori_loop(
        0, _K, step, (rowmax, z, z, zi, zi), unroll=2)
    score_ref[...] = sv[None]
    cls_ref[...] = cv[None]

    bxv, byv, wxv, wyv = z, z, z, z
    for t in range(_K):
        sel = lane_o == t
        rt = jnp.min(jnp.where(sel, rv, big))
        jt = jnp.min(jnp.where(sel, jv, big))
        box = y_ref[0, pl.ds(rt, 1), pl.ds(jt, 1), C:C + 4]
        bxv = jnp.where(sel, _SCALE * jt.astype(jnp.float32) + box[0, 0, 2], bxv)
        byv = jnp.where(sel, _SCALE * rt.astype(jnp.float32) + box[0, 0, 3], byv)
        wxv = jnp.where(sel, box[0, 0, 0], wxv)
        wyv = jnp.where(sel, box[0, 0, 1], wyv)
    bcx_ref[...] = bxv[None]
    bcy_ref[...] = byv[None]
    whx_ref[...] = (_SCALE * (jnp.exp(wxv) - 1.0))[None]
    why_ref[...] = (_SCALE * (jnp.exp(wyv) - 1.0))[None]


@jax.jit
def kernel(y):
    B, H, W, Ct = y.shape
    out_sds = jax.ShapeDtypeStruct((B, 1, 128), jnp.float32)
    outs = pl.pallas_call(
        _body,
        grid=(B,),
        in_specs=[
            pl.BlockSpec((1, H, W, Ct), lambda b: (b, 0, 0, 0)),
        ],
        out_specs=[pl.BlockSpec((1, 1, 128), lambda b: (b, 0, 0))] * 6,
        out_shape=[out_sds] * 6,
        scratch_shapes=[pltpu.VMEM((H, W, _C), jnp.float32)],
    )(y)
    sv, cv, bxv, byv, wxv, wyv = (o[:, 0, :] for o in outs)
    score_k = sv[:, :_K]
    classes = cv[:, :_K].astype(jnp.int32)
    bc_k = jnp.stack([bxv[:, :_K], byv[:, :_K]], axis=-1)
    wh_k = jnp.stack([wxv[:, :_K], wyv[:, :_K]], axis=-1)
    return (score_k, classes, bc_k, wh_k)


# speedup vs baseline: 1.2183x; 1.2183x over previous
import jax
import jax.numpy as jnp
from jax.experimental import pallas as pl
from jax.experimental.pallas import tpu as pltpu

_K = 100
_H = 160
_W = 160
_C = 80
_SCALE = 4.0


def _body(y_ref, score_ref, cls_ref, bcx_ref, bcy_ref,
          whx_ref, why_ref, keep_ref):
    H, W, C = _H, _W, _C
    ninf = jnp.float32(-jnp.inf)

    CH = 40
    pad_row = jnp.full((1, W, C), ninf, dtype=jnp.float32)
    pad_col = jnp.full((CH, 1, C), ninf, dtype=jnp.float32)
    rowmax_parts = [[], [], [], []]
    for c0 in range(0, H, CH):
        lo = max(c0 - 1, 0)
        hi = min(c0 + CH + 1, H)
        o = c0 - lo
        hmc = y_ref[0, lo:hi, :, :C]
        center = hmc[o:o + CH]
        if lo < c0:
            up = hmc[o - 1:o + CH - 1]
        else:
            up = jnp.concatenate([pad_row, hmc[0:CH - 1]], axis=0)
        if hi > c0 + CH:
            down = hmc[o + 1:o + CH + 1]
        else:
            down = jnp.concatenate([hmc[o + 1:o + CH], pad_row], axis=0)
        vmax = jnp.maximum(center, jnp.maximum(up, down))
        hmax = jnp.maximum(
            vmax,
            jnp.maximum(jnp.concatenate([vmax[:, 1:], pad_col], axis=1),
                        jnp.concatenate([pad_col, vmax[:, :-1]], axis=1)))
        keep_c = jnp.where(center == hmax, center, 0.0)
        keep_ref[c0:c0 + CH] = keep_c
        for q in range(4):
            rowmax_parts[q].append(
                jnp.max(jnp.max(keep_c[:, q * 40:(q + 1) * 40, :], axis=2),
                        axis=1).reshape(1, CH))
    qmax0 = jnp.concatenate(
        [jnp.concatenate(p, axis=1) for p in rowmax_parts], axis=1)

    QW = 40
    col_iota = (jax.lax.broadcasted_iota(jnp.int32, (QW, C), 0) * C
                + jax.lax.broadcasted_iota(jnp.int32, (QW, C), 1))
    lane_u = jax.lax.broadcasted_iota(jnp.int32, (1, H * 4), 1)
    prio = jnp.mod(lane_u, H) * 4 + lane_u // H
    lane_c = jax.lax.broadcasted_iota(jnp.int32, (1, 1, C), 2)
    lane_o = jax.lax.broadcasted_iota(jnp.int32, (1, 128), 1)
    big = jnp.int32(1 << 30)

    def step(t, carry):
        qmax, sv, cv, bxv, byv, wxv, wyv = carry
        m = jnp.max(qmax)
        u = jnp.min(jnp.where(qmax == m, prio, big))
        i = u // 4
        qo = (u % 4) * QW
        slab = keep_ref[pl.ds(i, 1), pl.ds(qo, QW)][0]
        cl = jnp.min(jnp.where(slab == m, col_iota, big))
        k = jnp.mod(cl, C)
        j = qo + cl // C
        new_slab = jnp.where(col_iota == cl, -1.0, slab)
        keep_ref[pl.ds(i, 1), pl.ds(j, 1)] = \
            jnp.where(lane_c == k, -1.0, keep_ref[pl.ds(i, 1), pl.ds(j, 1)])
        qmax = jnp.where(prio == u, jnp.max(new_slab), qmax)
        box = y_ref[0, pl.ds(i, 1), pl.ds(j, 1), C:C + 4]
        sel = lane_o == t
        sv = jnp.where(sel, m, sv)
        cv = jnp.where(sel, k.astype(jnp.float32), cv)
        bxv = jnp.where(sel, _SCALE * j.astype(jnp.float32) + box[0, 0, 2], bxv)
        byv = jnp.where(sel, _SCALE * i.astype(jnp.float32) + box[0, 0, 3], byv)
        wxv = jnp.where(sel, box[0, 0, 0], wxv)
        wyv = jnp.where(sel, box[0, 0, 1], wyv)
        return qmax, sv, cv, bxv, byv, wxv, wyv

    z = jnp.zeros((1, 128), jnp.float32)
    _, sv, cv, bxv, byv, wxv, wyv = jax.lax.fori_loop(
        0, _K, step, (qmax0, z, z, z, z, z, z), unroll=2)
    score_ref[...] = sv[None]
    cls_ref[...] = cv[None]
    bcx_ref[...] = bxv[None]
    bcy_ref[...] = byv[None]
    whx_ref[...] = (_SCALE * (jnp.exp(wxv) - 1.0))[None]
    why_ref[...] = (_SCALE * (jnp.exp(wyv) - 1.0))[None]


@jax.jit
def kernel(y):
    B, H, W, Ct = y.shape
    out_sds = jax.ShapeDtypeStruct((B, 1, 128), jnp.float32)
    outs = pl.pallas_call(
        _body,
        grid=(B,),
        in_specs=[
            pl.BlockSpec((1, H, W, Ct), lambda b: (b, 0, 0, 0)),
        ],
        out_specs=[pl.BlockSpec((1, 1, 128), lambda b: (b, 0, 0))] * 6,
        out_shape=[out_sds] * 6,
        scratch_shapes=[pltpu.VMEM((H, W, _C), jnp.float32)],
    )(y)
    sv, cv, bxv, byv, wxv, wyv = (o[:, 0, :] for o in outs)
    score_k = sv[:, :_K]
    classes = cv[:, :_K].astype(jnp.int32)
    bc_k = jnp.stack([bxv[:, :_K], byv[:, :_K]], axis=-1)
    wh_k = jnp.stack([wxv[:, :_K], wyv[:, :_K]], axis=-1)
    return (score_k, classes, bc_k, wh_k)


# speedup vs baseline: 1.6421x; 1.3478x over previous
import jax
import jax.numpy as jnp
from jax.experimental import pallas as pl
from jax.experimental.pallas import tpu as pltpu

_K = 100
_H = 160
_W = 160
_C = 80
_SCALE = 4.0


def _body(y_ref, score_ref, cls_ref, bcx_ref, bcy_ref,
          whx_ref, why_ref, keep_ref):
    H, W, C = _H, _W, _C
    ninf = jnp.float32(-jnp.inf)

    CH = 40
    pad_row = jnp.full((1, W, C), ninf, dtype=jnp.float32)
    pad_col = jnp.full((CH, 1, C), ninf, dtype=jnp.float32)
    rowmax_parts = [[], [], [], []]
    for c0 in range(0, H, CH):
        lo = max(c0 - 1, 0)
        hi = min(c0 + CH + 1, H)
        o = c0 - lo
        hmc = y_ref[0, lo:hi, :, :C]
        center = hmc[o:o + CH]
        if lo < c0:
            up = hmc[o - 1:o + CH - 1]
        else:
            up = jnp.concatenate([pad_row, hmc[0:CH - 1]], axis=0)
        if hi > c0 + CH:
            down = hmc[o + 1:o + CH + 1]
        else:
            down = jnp.concatenate([hmc[o + 1:o + CH], pad_row], axis=0)
        vmax = jnp.maximum(center, jnp.maximum(up, down))
        hmax = jnp.maximum(
            vmax,
            jnp.maximum(jnp.concatenate([vmax[:, 1:], pad_col], axis=1),
                        jnp.concatenate([pad_col, vmax[:, :-1]], axis=1)))
        keep_c = jnp.where(center == hmax, center, 0.0)
        keep_ref[c0:c0 + CH] = keep_c
        for q in range(4):
            rowmax_parts[q].append(
                jnp.max(jnp.max(keep_c[:, q * 40:(q + 1) * 40, :], axis=2),
                        axis=1).reshape(1, CH))
    qmax0 = jnp.concatenate(
        [jnp.concatenate(p, axis=1) for p in rowmax_parts], axis=1)

    QW = 40
    col_iota = (jax.lax.broadcasted_iota(jnp.int32, (QW, C), 0) * C
                + jax.lax.broadcasted_iota(jnp.int32, (QW, C), 1))
    lane_u = jax.lax.broadcasted_iota(jnp.int32, (1, H * 4), 1)
    prio = jnp.mod(lane_u, H) * 4 + lane_u // H
    lane_c = jax.lax.broadcasted_iota(jnp.int32, (1, 1, C), 2)
    lane_o = jax.lax.broadcasted_iota(jnp.int32, (1, 128), 1)
    big = jnp.int32(1 << 30)

    m0 = jnp.max(qmax0)
    u0 = jnp.min(jnp.where(qmax0 == m0, prio, big))
    slab0 = keep_ref[pl.ds(u0 // 4, 1), pl.ds((u0 % 4) * QW, QW)][0]

    def step(t, carry):
        qmax, u, m, slab, sv, cv, bxv, byv, wxv, wyv = carry
        i = u // 4
        qo = (u % 4) * QW
        not_u = prio != u
        m2 = jnp.max(jnp.where(not_u, qmax, -1.0))
        u2 = jnp.min(jnp.where((qmax == m2) & not_u, prio, big))
        slab2 = keep_ref[pl.ds(u2 // 4, 1), pl.ds((u2 % 4) * QW, QW)][0]
        cl = jnp.min(jnp.where(slab == m, col_iota, big))
        k = jnp.mod(cl, C)
        j = qo + cl // C
        new_slab = jnp.where(col_iota == cl, -1.0, slab)
        v_same = jnp.max(new_slab)
        keep_ref[pl.ds(i, 1), pl.ds(qo, QW)] = new_slab[None]
        qmax = jnp.where(prio == u, v_same, qmax)
        same = (v_same > m2) | ((v_same == m2) & (u < u2))
        un = jnp.where(same, u, u2)
        mn = jnp.where(same, v_same, m2)
        slabn = jnp.where(same, new_slab, slab2)
        box = y_ref[0, pl.ds(i, 1), pl.ds(j, 1), C:C + 4]
        sel = lane_o == t
        sv = jnp.where(sel, m, sv)
        cv = jnp.where(sel, k.astype(jnp.float32), cv)
        bxv = jnp.where(sel, _SCALE * j.astype(jnp.float32) + box[0, 0, 2], bxv)
        byv = jnp.where(sel, _SCALE * i.astype(jnp.float32) + box[0, 0, 3], byv)
        wxv = jnp.where(sel, box[0, 0, 0], wxv)
        wyv = jnp.where(sel, box[0, 0, 1], wyv)
        return qmax, un, mn, slabn, sv, cv, bxv, byv, wxv, wyv

    z = jnp.zeros((1, 128), jnp.float32)
    _, _, _, _, sv, cv, bxv, byv, wxv, wyv = jax.lax.fori_loop(
        0, _K, step, (qmax0, u0, m0, slab0, z, z, z, z, z, z), unroll=2)
    score_ref[...] = sv[None]
    cls_ref[...] = cv[None]
    bcx_ref[...] = bxv[None]
    bcy_ref[...] = byv[None]
    whx_ref[...] = (_SCALE * (jnp.exp(wxv) - 1.0))[None]
    why_ref[...] = (_SCALE * (jnp.exp(wyv) - 1.0))[None]


@jax.jit
def kernel(y):
    B, H, W, Ct = y.shape
    out_sds = jax.ShapeDtypeStruct((B, 1, 128), jnp.float32)
    outs = pl.pallas_call(
        _body,
        grid=(B,),
        in_specs=[
            pl.BlockSpec((1, H, W, Ct), lambda b: (b, 0, 0, 0)),
        ],
        out_specs=[pl.BlockSpec((1, 1, 128), lambda b: (b, 0, 0))] * 6,
        out_shape=[out_sds] * 6,
        scratch_shapes=[pltpu.VMEM((H, W, _C), jnp.float32)],
    )(y)
    sv, cv, bxv, byv, wxv, wyv = (o[:, 0, :] for o in outs)
    score_k = sv[:, :_K]
    classes = cv[:, :_K].astype(jnp.int32)
    bc_k = jnp.stack([bxv[:, :_K], byv[:, :_K]], axis=-1)
    wh_k = jnp.stack([wxv[:, :_K], wyv[:, :_K]], axis=-1)
    return (score_k, classes, bc_k, wh_k)


# speedup vs baseline: 1.6892x; 1.0287x over previous
import jax
import jax.numpy as jnp
from jax.experimental import pallas as pl
from jax.experimental.pallas import tpu as pltpu

_K = 100
_H = 160
_W = 160
_C = 80
_SCALE = 4.0


def _body(y_ref, score_ref, cls_ref, bcx_ref, bcy_ref,
          whx_ref, why_ref, keep_ref):
    H, W, C = _H, _W, _C
    ninf = jnp.float32(-jnp.inf)

    CH = 40
    pad_row = jnp.full((1, W, C), ninf, dtype=jnp.float32)
    pad_col = jnp.full((CH, 1, C), ninf, dtype=jnp.float32)
    rowmax_parts = [[], [], [], []]
    for c0 in range(0, H, CH):
        lo = max(c0 - 1, 0)
        hi = min(c0 + CH + 1, H)
        o = c0 - lo
        hmc = y_ref[0, lo:hi, :, :C]
        center = hmc[o:o + CH]
        if lo < c0:
            up = hmc[o - 1:o + CH - 1]
        else:
            up = jnp.concatenate([pad_row, hmc[0:CH - 1]], axis=0)
        if hi > c0 + CH:
            down = hmc[o + 1:o + CH + 1]
        else:
            down = jnp.concatenate([hmc[o + 1:o + CH], pad_row], axis=0)
        vmax = jnp.maximum(center, jnp.maximum(up, down))
        hmax = jnp.maximum(
            vmax,
            jnp.maximum(jnp.concatenate([vmax[:, 1:], pad_col], axis=1),
                        jnp.concatenate([pad_col, vmax[:, :-1]], axis=1)))
        keep_c = jnp.where(center == hmax, center, 0.0)
        keep_ref[c0:c0 + CH] = keep_c
        for q in range(4):
            rowmax_parts[q].append(
                jnp.max(jnp.max(keep_c[:, q * 40:(q + 1) * 40, :], axis=2),
                        axis=1).reshape(1, CH))
    qmax0 = jnp.concatenate(
        [jnp.concatenate(p, axis=1) for p in rowmax_parts], axis=1)

    QW = 40
    col_iota = (jax.lax.broadcasted_iota(jnp.int32, (QW, C), 0) * C
                + jax.lax.broadcasted_iota(jnp.int32, (QW, C), 1))
    lane_u = jax.lax.broadcasted_iota(jnp.int32, (1, H * 4), 1)
    prio = jnp.mod(lane_u, H) * 4 + lane_u // H
    lane_c = jax.lax.broadcasted_iota(jnp.int32, (1, 1, C), 2)
    lane_o = jax.lax.broadcasted_iota(jnp.int32, (1, 128), 1)
    big = jnp.int32(1 << 30)

    m0 = jnp.max(qmax0)
    u0 = jnp.min(jnp.where(qmax0 == m0, prio, big))
    slab0 = keep_ref[pl.ds(u0 // 4, 1), pl.ds((u0 % 4) * QW, QW)][0]

    def step(t, carry):
        qmax, u, m, slab, sv, cv, bxv, byv, wxv, wyv = carry
        i = u // 4
        qo = (u % 4) * QW
        not_u = prio != u
        m2 = jnp.max(jnp.where(not_u, qmax, -1.0))
        u2 = jnp.min(jnp.where((qmax == m2) & not_u, prio, big))
        slab2 = keep_ref[pl.ds(u2 // 4, 1), pl.ds((u2 % 4) * QW, QW)][0]
        cl = jnp.min(jnp.where(slab == m, col_iota, big))
        k = jnp.mod(cl, C)
        j = qo + cl // C
        new_slab = jnp.where(col_iota == cl, -1.0, slab)
        v_same = jnp.max(new_slab)
        keep_ref[pl.ds(i, 1), pl.ds(qo, QW)] = new_slab[None]
        qmax = jnp.where(prio == u, v_same, qmax)
        same = (v_same > m2) | ((v_same == m2) & (u < u2))
        un = jnp.where(same, u, u2)
        mn = jnp.where(same, v_same, m2)
        slabn = jnp.where(same, new_slab, slab2)
        box = y_ref[0, pl.ds(i, 1), pl.ds(j, 1), C:C + 4]
        sel = lane_o == t
        sv = jnp.where(sel, m, sv)
        cv = jnp.where(sel, k.astype(jnp.float32), cv)
        bxv = jnp.where(sel, _SCALE * j.astype(jnp.float32) + box[0, 0, 2], bxv)
        byv = jnp.where(sel, _SCALE * i.astype(jnp.float32) + box[0, 0, 3], byv)
        wxv = jnp.where(sel, box[0, 0, 0], wxv)
        wyv = jnp.where(sel, box[0, 0, 1], wyv)
        return qmax, un, mn, slabn, sv, cv, bxv, byv, wxv, wyv

    z = jnp.zeros((1, 128), jnp.float32)
    _, _, _, _, sv, cv, bxv, byv, wxv, wyv = jax.lax.fori_loop(
        0, _K, step, (qmax0, u0, m0, slab0, z, z, z, z, z, z), unroll=4)
    score_ref[...] = sv[None]
    cls_ref[...] = cv[None]
    bcx_ref[...] = bxv[None]
    bcy_ref[...] = byv[None]
    whx_ref[...] = (_SCALE * (jnp.exp(wxv) - 1.0))[None]
    why_ref[...] = (_SCALE * (jnp.exp(wyv) - 1.0))[None]


@jax.jit
def kernel(y):
    B, H, W, Ct = y.shape
    out_sds = jax.ShapeDtypeStruct((B, 1, 128), jnp.float32)
    outs = pl.pallas_call(
        _body,
        grid=(B,),
        in_specs=[
            pl.BlockSpec((1, H, W, Ct), lambda b: (b, 0, 0, 0)),
        ],
        out_specs=[pl.BlockSpec((1, 1, 128), lambda b: (b, 0, 0))] * 6,
        out_shape=[out_sds] * 6,
        scratch_shapes=[pltpu.VMEM((H, W, _C), jnp.float32)],
    )(y)
    sv, cv, bxv, byv, wxv, wyv = (o[:, 0, :] for o in outs)
    score_k = sv[:, :_K]
    classes = cv[:, :_K].astype(jnp.int32)
    bc_k = jnp.stack([bxv[:, :_K], byv[:, :_K]], axis=-1)
    wh_k = jnp.stack([wxv[:, :_K], wyv[:, :_K]], axis=-1)
    return (score_k, classes, bc_k, wh_k)


# speedup vs baseline: 1.6988x; 1.0057x over previous
import jax
import jax.numpy as jnp
from jax.experimental import pallas as pl
from jax.experimental.pallas import tpu as pltpu

_K = 100
_H = 160
_W = 160
_C = 80
_SCALE = 4.0


def _body(y_ref, score_ref, cls_ref, bcx_ref, bcy_ref,
          whx_ref, why_ref, keep_ref):
    H, W, C = _H, _W, _C
    ninf = jnp.float32(-jnp.inf)

    CH = 40
    pad_row = jnp.full((1, W, C), ninf, dtype=jnp.float32)
    pad_col = jnp.full((CH, 1, C), ninf, dtype=jnp.float32)
    rowmax_parts = [[], [], [], []]
    for c0 in range(0, H, CH):
        lo = max(c0 - 1, 0)
        hi = min(c0 + CH + 1, H)
        o = c0 - lo
        hmc = y_ref[0, lo:hi, :, :C]
        center = hmc[o:o + CH]
        if lo < c0:
            up = hmc[o - 1:o + CH - 1]
        else:
            up = jnp.concatenate([pad_row, hmc[0:CH - 1]], axis=0)
        if hi > c0 + CH:
            down = hmc[o + 1:o + CH + 1]
        else:
            down = jnp.concatenate([hmc[o + 1:o + CH], pad_row], axis=0)
        vmax = jnp.maximum(center, jnp.maximum(up, down))
        hmax = jnp.maximum(
            vmax,
            jnp.maximum(jnp.concatenate([vmax[:, 1:], pad_col], axis=1),
                        jnp.concatenate([pad_col, vmax[:, :-1]], axis=1)))
        keep_c = jnp.where(center == hmax, center, 0.0)
        keep_ref[c0:c0 + CH] = keep_c
        for q in range(4):
            rowmax_parts[q].append(
                jnp.max(jnp.max(keep_c[:, q * 40:(q + 1) * 40, :], axis=2),
                        axis=1).reshape(1, CH))
    qmax0 = jnp.concatenate(
        [jnp.concatenate(p, axis=1) for p in rowmax_parts], axis=1)

    QW = 40
    col_iota = (jax.lax.broadcasted_iota(jnp.int32, (QW, C), 0) * C
                + jax.lax.broadcasted_iota(jnp.int32, (QW, C), 1))
    lane_u = jax.lax.broadcasted_iota(jnp.int32, (1, H * 4), 1)
    prio = jnp.mod(lane_u, H) * 4 + lane_u // H
    lane_c = jax.lax.broadcasted_iota(jnp.int32, (1, 1, C), 2)
    lane_o = jax.lax.broadcasted_iota(jnp.int32, (1, 128), 1)
    big = jnp.int32(1 << 30)

    m0 = jnp.max(qmax0)
    u0 = jnp.min(jnp.where(qmax0 == m0, prio, big))
    slab0 = keep_ref[pl.ds(u0 // 4, 1), pl.ds((u0 % 4) * QW, QW)][0]

    def step(t, carry):
        qmax, u, m, slab, sv, cv, bxv, byv, wxv, wyv = carry
        i = u // 4
        qo = (u % 4) * QW
        not_u = prio != u
        m2 = jnp.max(jnp.where(not_u, qmax, -1.0))
        u2 = jnp.min(jnp.where((qmax == m2) & not_u, prio, big))
        slab2 = keep_ref[pl.ds(u2 // 4, 1), pl.ds((u2 % 4) * QW, QW)][0]
        cl = jnp.min(jnp.where(slab == m, col_iota, big))
        k = jnp.mod(cl, C)
        j = qo + cl // C
        new_slab = jnp.where(col_iota == cl, -1.0, slab)
        v_same = jnp.max(new_slab)
        keep_ref[pl.ds(i, 1), pl.ds(qo, QW)] = new_slab[None]
        qmax = jnp.where(prio == u, v_same, qmax)
        same = (v_same > m2) | ((v_same == m2) & (u < u2))
        un = jnp.where(same, u, u2)
        mn = jnp.where(same, v_same, m2)
        slabn = jnp.where(same, new_slab, slab2)
        box = y_ref[0, pl.ds(i, 1), pl.ds(j, 1), C:C + 4]
        sel = lane_o == t
        sv = jnp.where(sel, m, sv)
        cv = jnp.where(sel, k.astype(jnp.float32), cv)
        bxv = jnp.where(sel, _SCALE * j.astype(jnp.float32) + box[0, 0, 2], bxv)
        byv = jnp.where(sel, _SCALE * i.astype(jnp.float32) + box[0, 0, 3], byv)
        wxv = jnp.where(sel, box[0, 0, 0], wxv)
        wyv = jnp.where(sel, box[0, 0, 1], wyv)
        return qmax, un, mn, slabn, sv, cv, bxv, byv, wxv, wyv

    z = jnp.zeros((1, 128), jnp.float32)
    _, _, _, _, sv, cv, bxv, byv, wxv, wyv = jax.lax.fori_loop(
        0, _K, step, (qmax0, u0, m0, slab0, z, z, z, z, z, z), unroll=5)
    score_ref[...] = sv[None]
    cls_ref[...] = cv[None]
    bcx_ref[...] = bxv[None]
    bcy_ref[...] = byv[None]
    whx_ref[...] = (_SCALE * (jnp.exp(wxv) - 1.0))[None]
    why_ref[...] = (_SCALE * (jnp.exp(wyv) - 1.0))[None]


@jax.jit
def kernel(y):
    B, H, W, Ct = y.shape
    out_sds = jax.ShapeDtypeStruct((B, 1, 128), jnp.float32)
    outs = pl.pallas_call(
        _body,
        grid=(B,),
        in_specs=[
            pl.BlockSpec((1, H, W, Ct), lambda b: (b, 0, 0, 0)),
        ],
        out_specs=[pl.BlockSpec((1, 1, 128), lambda b: (b, 0, 0))] * 6,
        out_shape=[out_sds] * 6,
        scratch_shapes=[pltpu.VMEM((H, W, _C), jnp.float32)],
    )(y)
    sv, cv, bxv, byv, wxv, wyv = (o[:, 0, :] for o in outs)
    score_k = sv[:, :_K]
    classes = cv[:, :_K].astype(jnp.int32)
    bc_k = jnp.stack([bxv[:, :_K], byv[:, :_K]], axis=-1)
    wh_k = jnp.stack([wxv[:, :_K], wyv[:, :_K]], axis=-1)
    return (score_k, classes, bc_k, wh_k)
